# baseline (speedup 1.0000x reference)
import jax
import jax.numpy as jnp
from jax import lax
from jax.experimental import pallas as pl
from jax.experimental.pallas import tpu as pltpu
from jax.experimental.pallas import tpu_sc as plsc

NUM_ENTITIES = 1000000
NUM_RELATIONS = 1000
D = 128
D2 = 64
B = 16384
NC = 2
NS = 16
NW = NC * NS
BPW = B // NW
CHUNK = 64
NCHUNK = BPW // CHUNK
L = 16


def _trig_body(rel_ref, rot_ref):
    ph = rel_ref[...]
    rot_ref[...] = jnp.concatenate([jnp.cos(ph), jnp.sin(ph)], axis=1)


def _make_rot_table(relation_emb):
    return pl.pallas_call(
        _trig_body,
        out_shape=jax.ShapeDtypeStruct((NUM_RELATIONS, D), jnp.float32),
    )(relation_emb)


def _sqrt16(x):
    i = plsc.bitcast(x, jnp.int32)
    y = plsc.bitcast(jnp.int32(0x5F3759DF) - (i >> 1), jnp.float32)
    y = y * (1.5 - (x * 0.5) * y * y)
    return x * y


def _score_chunk(h_v, t_v, r_v, part_v, out_v, boff, c):
    def row_body(r, _):
        rb = boff + r
        acc = jnp.zeros((L,), jnp.float32)
        for g in range(D2 // L):
            lo = g * L
            hi = D2 + g * L
            reh = h_v[rb, pl.ds(lo, L)]
            imh = h_v[rb, pl.ds(hi, L)]
            ret = t_v[rb, pl.ds(lo, L)]
            imt = t_v[rb, pl.ds(hi, L)]
            cr = r_v[rb, pl.ds(lo, L)]
            sr = r_v[rb, pl.ds(hi, L)]
            re_s = reh * cr - imh * sr - ret
            im_s = reh * sr + imh * cr - imt
            acc = acc + _sqrt16(re_s * re_s + im_s * im_s)
        part_v[r, :] = acc
        return 0

    lax.fori_loop(0, CHUNK, row_body, 0)

    def blk_body(blk, _):
        rows = blk * L + lax.iota(jnp.int32, L)
        acc16 = jnp.zeros((L,), jnp.float32)
        for k in range(L):
            cols = jnp.full((L,), k, jnp.int32)
            acc16 = acc16 + plsc.load_gather(part_v, [rows, cols])
        out_v[pl.ds(c * CHUNK + blk * L, L)] = -acc16
        return 0

    lax.fori_loop(0, CHUNK // L, blk_body, 0)


def _sc_body(h_idx_hbm, t_idx_hbm, r_idx_hbm, ent_hbm, rot_hbm,
             out_hbm, idxh_v, idxt_v, idxr_v, hbuf, tbuf, rbuf,
             part_v, out_v, sem0, sem1):
    wid = lax.axis_index("s") * NC + lax.axis_index("c")
    base = wid * BPW
    ci0 = pltpu.async_copy(h_idx_hbm.at[pl.ds(base, BPW)], idxh_v, sem0)
    ci1 = pltpu.async_copy(t_idx_hbm.at[pl.ds(base, BPW)], idxt_v, sem0)
    ci2 = pltpu.async_copy(r_idx_hbm.at[pl.ds(base, BPW)], idxr_v, sem0)
    ci0.wait()
    ci1.wait()
    ci2.wait()

    def fire(c, b):
        sl = pl.ds(c * CHUNK, CHUNK)
        dsl = pl.ds(b * CHUNK, CHUNK)
        sem = (sem0, sem1)[b]
        pltpu.async_copy(ent_hbm.at[idxh_v.at[sl]], hbuf.at[dsl], sem)
        pltpu.async_copy(ent_hbm.at[idxt_v.at[sl]], tbuf.at[dsl], sem)
        pltpu.async_copy(rot_hbm.at[idxr_v.at[sl]], rbuf.at[dsl], sem)

    def drain(b):
        dummy = ent_hbm.at[pl.ds(0, CHUNK)]
        dsl = pl.ds(b * CHUNK, CHUNK)
        sem = (sem0, sem1)[b]
        for dst in (hbuf.at[dsl], tbuf.at[dsl], rbuf.at[dsl]):
            pltpu.make_async_copy(dummy, dst, sem).wait()

    fire(0, 0)

    def chunk_body(c, _):
        par = lax.rem(c, 2)

        @pl.when(jnp.logical_and(par == 0, c + 1 < NCHUNK))
        def _():
            fire(c + 1, 1)

        @pl.when(jnp.logical_and(par == 1, c + 1 < NCHUNK))
        def _():
            fire(c + 1, 0)

        @pl.when(par == 0)
        def _():
            drain(0)

        @pl.when(par == 1)
        def _():
            drain(1)

        boff = par * CHUNK
        _score_chunk(hbuf, tbuf, rbuf, part_v, out_v, boff, c)
        return 0

    lax.fori_loop(0, NCHUNK, chunk_body, 0)
    pltpu.sync_copy(out_v, out_hbm.at[pl.ds(base, BPW)])


@jax.jit
def _rotate_score(h_idx, r_idx, t_idx, entity_emb, rot_table):
    mesh = plsc.VectorSubcoreMesh(core_axis_name="c", subcore_axis_name="s",
                                  num_cores=NC, num_subcores=NS)
    return pl.kernel(
        _sc_body,
        out_type=jax.ShapeDtypeStruct((B,), jnp.float32),
        mesh=mesh,
        compiler_params=pltpu.CompilerParams(needs_layout_passes=False),
        scratch_types=[
            pltpu.VMEM((BPW,), jnp.int32),
            pltpu.VMEM((BPW,), jnp.int32),
            pltpu.VMEM((BPW,), jnp.int32),
            pltpu.VMEM((2 * CHUNK, D), jnp.float32),
            pltpu.VMEM((2 * CHUNK, D), jnp.float32),
            pltpu.VMEM((2 * CHUNK, D), jnp.float32),
            pltpu.VMEM((CHUNK, L), jnp.float32),
            pltpu.VMEM((BPW,), jnp.float32),
            pltpu.SemaphoreType.DMA,
            pltpu.SemaphoreType.DMA,
        ],
    )(h_idx, t_idx, r_idx, entity_emb, rot_table)


def kernel(h_idx, r_idx, t_idx, entity_emb, relation_emb):
    rot_table = _make_rot_table(relation_emb)
    return _rotate_score(h_idx, r_idx, t_idx, entity_emb, rot_table)


# speedup vs baseline: 1.1642x; 1.1642x over previous
import jax
import jax.numpy as jnp
from jax import lax
from jax.experimental import pallas as pl
from jax.experimental.pallas import tpu as pltpu
from jax.experimental.pallas import tpu_sc as plsc

NUM_ENTITIES = 1000000
NUM_RELATIONS = 1000
D = 128
D2 = 64
B = 16384
NC = 2
NS = 16
NW = NC * NS
BPW = B // NW
CHUNK = 64
NCHUNK = BPW // CHUNK
L = 16


def _trig_body(rel_ref, rot_ref):
    ph = rel_ref[...]
    rot_ref[...] = jnp.concatenate([jnp.cos(ph), jnp.sin(ph)], axis=1)


def _make_rot_table(relation_emb):
    return pl.pallas_call(
        _trig_body,
        out_shape=jax.ShapeDtypeStruct((NUM_RELATIONS, D), jnp.float32),
    )(relation_emb)


def _sqrt16_x2(x):
    i = plsc.bitcast(x, jnp.int32)
    y = plsc.bitcast(jnp.int32(0x5F3759DF) - (i >> 1), jnp.float32)
    return x * (y * (3.0 - x * y * y))


def _score_chunk(h_v, t_v, r_v, part_v, out_v, c):
    def row_body(r, _):
        acc = jnp.zeros((L,), jnp.float32)
        for g in range(D2 // L):
            lo = g * L
            hi = D2 + g * L
            reh = h_v[r, pl.ds(lo, L)]
            imh = h_v[r, pl.ds(hi, L)]
            ret = t_v[r, pl.ds(lo, L)]
            imt = t_v[r, pl.ds(hi, L)]
            cr = r_v[r, pl.ds(lo, L)]
            sr = r_v[r, pl.ds(hi, L)]
            re_s = reh * cr - imh * sr - ret
            im_s = reh * sr + imh * cr - imt
            acc = acc + _sqrt16_x2(re_s * re_s + im_s * im_s)
        part_v[r, :] = acc
        return 0

    lax.fori_loop(0, CHUNK, row_body, 0)

    def blk_body(blk, _):
        rows = blk * L + lax.iota(jnp.int32, L)
        acc16 = jnp.zeros((L,), jnp.float32)
        for k in range(L):
            cols = jnp.full((L,), k, jnp.int32)
            acc16 = acc16 + plsc.load_gather(part_v, [rows, cols])
        out_v[pl.ds(c * CHUNK + blk * L, L)] = acc16 * (-0.5)
        return 0

    lax.fori_loop(0, CHUNK // L, blk_body, 0)


def _sc_body(h_idx_hbm, t_idx_hbm, r_idx_hbm, ent_hbm, rot_hbm,
             out_hbm, idxh_v, idxt_v, idxr_v, hbuf, tbuf, rbuf,
             part_v, out_v, sem0, sem1):
    wid = lax.axis_index("s") * NC + lax.axis_index("c")
    base = wid * BPW
    ci0 = pltpu.async_copy(h_idx_hbm.at[pl.ds(base, BPW)], idxh_v, sem0)
    ci1 = pltpu.async_copy(t_idx_hbm.at[pl.ds(base, BPW)], idxt_v, sem0)
    ci2 = pltpu.async_copy(r_idx_hbm.at[pl.ds(base, BPW)], idxr_v, sem0)
    ci0.wait()
    ci1.wait()
    ci2.wait()
    sems = (sem0, sem1)
    bufs = ((hbuf.at[0], tbuf.at[0], rbuf.at[0]),
            (hbuf.at[1], tbuf.at[1], rbuf.at[1]))

    def fire(c, b):
        sl = pl.ds(c * CHUNK, CHUNK)
        h_b, t_b, r_b = bufs[b]
        pltpu.async_copy(ent_hbm.at[idxh_v.at[sl]], h_b, sems[b])
        pltpu.async_copy(ent_hbm.at[idxt_v.at[sl]], t_b, sems[b])
        pltpu.async_copy(rot_hbm.at[idxr_v.at[sl]], r_b, sems[b])

    def drain(b):
        dummy = ent_hbm.at[pl.ds(0, CHUNK)]
        for dst in bufs[b]:
            pltpu.make_async_copy(dummy, dst, sems[b]).wait()

    fire(0, 0)

    def chunk_pair(i, _):
        for b in (0, 1):
            c = i * 2 + b

            @pl.when(c + 1 < NCHUNK)
            def _():
                fire(c + 1, 1 - b)

            drain(b)
            h_b, t_b, r_b = bufs[b]
            _score_chunk(h_b, t_b, r_b, part_v, out_v, c)
        return 0

    lax.fori_loop(0, NCHUNK // 2, chunk_pair, 0)
    pltpu.sync_copy(out_v, out_hbm.at[pl.ds(base, BPW)])


@jax.jit
def _rotate_score(h_idx, r_idx, t_idx, entity_emb, rot_table):
    mesh = plsc.VectorSubcoreMesh(core_axis_name="c", subcore_axis_name="s",
                                  num_cores=NC, num_subcores=NS)
    return pl.kernel(
        _sc_body,
        out_type=jax.ShapeDtypeStruct((B,), jnp.float32),
        mesh=mesh,
        compiler_params=pltpu.CompilerParams(needs_layout_passes=False),
        scratch_types=[
            pltpu.VMEM((BPW,), jnp.int32),
            pltpu.VMEM((BPW,), jnp.int32),
            pltpu.VMEM((BPW,), jnp.int32),
            pltpu.VMEM((2, CHUNK, D), jnp.float32),
            pltpu.VMEM((2, CHUNK, D), jnp.float32),
            pltpu.VMEM((2, CHUNK, D), jnp.float32),
            pltpu.VMEM((CHUNK, L), jnp.float32),
            pltpu.VMEM((BPW,), jnp.float32),
            pltpu.SemaphoreType.DMA,
            pltpu.SemaphoreType.DMA,
        ],
    )(h_idx, t_idx, r_idx, entity_emb, rot_table)


def kernel(h_idx, r_idx, t_idx, entity_emb, relation_emb):
    rot_table = _make_rot_table(relation_emb)
    return _rotate_score(h_idx, r_idx, t_idx, entity_emb, rot_table)
